# baseline (speedup 1.0000x reference)
import functools

import jax
import jax.numpy as jnp
from jax import lax
from jax.experimental import pallas as pl
from jax.experimental.pallas import tpu as pltpu
from jax.experimental.pallas import tpu_sc as plsc

EMBED = 64
CHUNK = 128
K = 4
LANES = 16
N_WORKERS = 32


@functools.lru_cache(maxsize=None)
def _build_transpose(embed: int, vocab: int):
    mesh = plsc.VectorSubcoreMesh(core_axis_name="c", subcore_axis_name="s")
    n_full = vocab // CHUNK
    rem = vocab % CHUNK
    out_rows = n_full * CHUNK + (CHUNK if rem else 0)

    SUP = 4
    assert n_full % SUP == 0
    n_sup = n_full // SUP

    PITCH = SUP * CHUNK + 17

    @functools.partial(
        pl.kernel,
        mesh=mesh,
        out_type=jax.ShapeDtypeStruct((out_rows * 2 * EMBED,), jnp.float32),
        scratch_types=[
            pltpu.VMEM((embed, PITCH), jnp.float32),
            pltpu.VMEM((embed, PITCH), jnp.float32),
            pltpu.VMEM((CHUNK * 2 * EMBED,), jnp.float32),
            pltpu.VMEM((CHUNK * 2 * EMBED,), jnp.float32),
            pltpu.SemaphoreType.DMA,
            pltpu.SemaphoreType.DMA,
        ],
        compiler_params=pltpu.CompilerParams(
            use_tc_tiling_on_sc=True, needs_layout_passes=False
        ),
    )
    def transpose_kernel(
        tt_hbm, tail_hbm, out_hbm, tv0, tv1, outb0, outb1, isem, osem
    ):
        tiles = (tv0, tv1)
        outb = (outb0, outb1)
        nc = lax.axis_size("c")
        wid = lax.axis_index("s") * nc + lax.axis_index("c")
        blk_words = CHUNK * 2 * EMBED
        e_ids = [
            lax.iota(jnp.int32, LANES) + LANES * c4
            for c4 in range(embed // LANES)
        ]

        def in_desc(j, b):
            v0 = (wid + N_WORKERS * j) * SUP * CHUNK
            return pltpu.make_async_copy(
                tt_hbm.at[:, pl.ds(v0, SUP * CHUNK)],
                tiles[b].at[:, pl.ds(0, SUP * CHUNK)],
                isem,
            )

        def out_desc(j, sb, ob):
            w0 = ((wid + N_WORKERS * j) * SUP + sb) * blk_words
            return pltpu.make_async_copy(
                outb[ob], out_hbm.at[pl.ds(w0, blk_words)], osem
            )

        n_mine = n_sup // N_WORKERS + jnp.where(
            wid < n_sup % N_WORKERS, 1, 0
        )
        pl.when(n_mine > 0)(lambda: in_desc(0, 0).start())

        def process(j, b):
            pl.when(j + 1 < n_mine)(lambda: in_desc(j + 1, 1 - b).start())
            in_desc(j, b).wait()
            for sb in range(SUP):
                ob = sb % 2
                if sb >= 2:
                    out_desc(j, sb - 2, ob).wait()
                else:
                    pl.when(j >= 1)(
                        lambda sb=sb, ob=ob: out_desc(
                            j - 1, sb + SUP - 2, ob
                        ).wait()
                    )

                def body_j(jj, carry, sb=sb, ob=ob):
                    col = jnp.zeros((LANES,), jnp.int32) + (sb * CHUNK + jj)
                    vecs = [
                        plsc.load_gather(tiles[b], [e_id, col])
                        for e_id in e_ids
                    ]
                    for c4 in range(embed // LANES):
                        outb[ob][pl.ds(jj * (2 * EMBED) + LANES * c4, LANES)] = (
                            vecs[c4]
                        )
                    return carry

                lax.fori_loop(0, CHUNK, body_j, 0)
                out_desc(j, sb, ob).start()

        def do_pair(i2, carry):
            for b in (0, 1):
                j = 2 * i2 + b
                pl.when(j < n_mine)(lambda j=j, b=b: process(j, b))
            return carry

        lax.fori_loop(0, (n_mine + 1) // 2, do_pair, 0)

        def drain(i):
            pl.when(i >= 0)(lambda: out_desc(0, 0, 0).wait())

        drain(n_mine - 1)
        drain(n_mine - 1)

        if rem:
            @pl.when(wid == N_WORKERS - 1)
            def _():
                pltpu.sync_copy(tail_hbm, outb0)
                pltpu.sync_copy(
                    outb0, out_hbm.at[pl.ds(n_full * blk_words, blk_words)]
                )

    return transpose_kernel


@functools.lru_cache(maxsize=None)
def _build_gather(seq: int, batch: int, table_rows: int):
    mesh = plsc.VectorSubcoreMesh(core_axis_name="c", subcore_axis_name="s")
    n_chunks = seq
    assert n_chunks % K == 0
    n_blocks = n_chunks // K
    blk = K * CHUNK

    @functools.partial(
        pl.kernel,
        mesh=mesh,
        out_type=jax.ShapeDtypeStruct((seq * batch, 2 * EMBED), jnp.float32),
        scratch_types=[
            pltpu.VMEM((n_chunks, CHUNK), jnp.int32),
            pltpu.VMEM((2, blk // 2, 2 * EMBED), jnp.float32),
            pltpu.SemaphoreType.DMA,
            pltpu.SemaphoreType.DMA,
        ],
        compiler_params=pltpu.CompilerParams(use_tc_tiling_on_sc=True),
    )
    def gather_kernel(table_hbm, idx_hbm, out_hbm, idx_v, rows_v, gsem, osem):
        nc = lax.axis_size("c")
        wid = lax.axis_index("s") * nc + lax.axis_index("c")
        col0 = wid * CHUNK
        pltpu.sync_copy(idx_hbm.at[:, pl.ds(col0, CHUNK)], idx_v)

        def fire(t, b):
            for k in range(K // 2):
                pltpu.make_async_copy(
                    table_hbm.at[idx_v.at[t * (K // 2) + k]],
                    rows_v.at[b, pl.ds(k * CHUNK, CHUNK)],
                    gsem,
                ).start()

        def wait_gathers(b):
            for k in range(K // 2):
                pltpu.make_async_copy(
                    table_hbm.at[idx_v.at[k]],
                    rows_v.at[b, pl.ds(k * CHUNK, CHUNK)],
                    gsem,
                ).wait()

        def out_copies(t, b):
            return [
                pltpu.make_async_copy(
                    rows_v.at[b, pl.ds(k * CHUNK, CHUNK)],
                    out_hbm.at[
                        pl.ds((t * (K // 2) + k) * batch + col0, CHUNK)
                    ],
                    osem,
                )
                for k in range(K // 2)
            ]

        n_b = n_chunks // (K // 2)
        fire(0, 0)

        def step(t, carry):
            b = lax.rem(t, 2)

            def drain_prev():
                for c in out_copies(t - 1, 1 - b):
                    c.wait()

            pl.when(t >= 1)(drain_prev)
            pl.when(t < n_b - 1)(lambda: fire(t + 1, 1 - b))
            wait_gathers(b)
            for c in out_copies(t, b):
                c.start()
            return carry

        lax.fori_loop(0, n_b, step, 0)
        for c in out_copies(n_b - 1, (n_b - 1) % 2):
            c.wait()

    return gather_kernel


def kernel(x, table):
    b, s = x.shape
    vocab, embed = table.shape
    assert embed == EMBED
    xt = jnp.swapaxes(x, 0, 1).astype(jnp.int32)
    tt = jnp.swapaxes(table, 0, 1)
    rem = vocab % CHUNK
    if rem:
        tail = jnp.pad(
            table[vocab - rem:, :], ((0, CHUNK - rem), (0, 2 * EMBED - embed))
        )
    else:
        tail = jnp.zeros((CHUNK, 2 * EMBED), jnp.float32)
    tflat = _build_transpose(embed, vocab)(tt, tail.reshape(-1))
    table128 = tflat.reshape(-1, 2 * EMBED)
    out = _build_gather(s, b, table128.shape[0])(table128, xt)
    return jnp.swapaxes(out[:, :embed].reshape(s, b, embed), 0, 1)


# speedup vs baseline: 1.9265x; 1.9265x over previous
import functools

import jax
import jax.numpy as jnp
from jax import lax
from jax.experimental import pallas as pl
from jax.experimental.pallas import tpu as pltpu
from jax.experimental.pallas import tpu_sc as plsc

EMBED = 64
CHUNK = 128
K = 4


@functools.lru_cache(maxsize=None)
def _build_gather(seq: int, batch: int, vocab: int):
    mesh = plsc.VectorSubcoreMesh(core_axis_name="c", subcore_axis_name="s")
    n_workers = 32
    assert batch % (n_workers * CHUNK) == 0 or batch == n_workers * CHUNK
    n_chunks = seq
    assert n_chunks % K == 0
    n_blocks = n_chunks // K
    blk = K * CHUNK

    @functools.partial(
        pl.kernel,
        mesh=mesh,
        out_type=jax.ShapeDtypeStruct((seq * batch, 2 * EMBED), jnp.float32),
        scratch_types=[
            pltpu.VMEM((n_chunks, CHUNK), jnp.int32),
            pltpu.VMEM((2, blk, EMBED), jnp.float32),
            pltpu.SemaphoreType.DMA,
            pltpu.SemaphoreType.DMA,
        ],
        compiler_params=pltpu.CompilerParams(use_tc_tiling_on_sc=False),
    )
    def gather_kernel(table_hbm, idx_hbm, out_hbm, idx_v, rows_v, gsem, osem):
        nc = lax.axis_size("c")
        wid = lax.axis_index("s") * nc + lax.axis_index("c")
        col0 = wid * CHUNK
        pltpu.sync_copy(idx_hbm.at[:, pl.ds(col0, CHUNK)], idx_v)

        def fire(t, b):
            for k in range(K):
                pltpu.make_async_copy(
                    table_hbm.at[idx_v.at[t * K + k]],
                    rows_v.at[b, pl.ds(k * CHUNK, CHUNK)],
                    gsem,
                ).start()

        def wait_gathers(b):
            for k in range(K):
                pltpu.make_async_copy(
                    table_hbm.at[idx_v.at[k]],
                    rows_v.at[b, pl.ds(k * CHUNK, CHUNK)],
                    gsem,
                ).wait()

        def out_copies(t, b):
            return [
                pltpu.make_async_copy(
                    rows_v.at[b, pl.ds(k * CHUNK, CHUNK)],
                    out_hbm.at[
                        pl.ds((t * K + k) * batch + col0, CHUNK),
                        pl.ds(0, EMBED),
                    ],
                    osem,
                )
                for k in range(K)
            ]

        fire(0, 0)

        def step(t, carry):
            b = lax.rem(t, 2)
            def drain_prev():
                for c in out_copies(t - 1, 1 - b):
                    c.wait()

            pl.when(t >= 1)(drain_prev)
            pl.when(t < n_blocks - 1)(lambda: fire(t + 1, 1 - b))
            wait_gathers(b)
            for c in out_copies(t, b):
                c.start()
            return carry

        lax.fori_loop(0, n_blocks, step, 0)
        for c in out_copies(n_blocks - 1, (n_blocks - 1) % 2):
            c.wait()

    return gather_kernel


def kernel(x, table):
    b, s = x.shape
    vocab, embed = table.shape
    assert embed == EMBED
    xt = jnp.swapaxes(x, 0, 1).astype(jnp.int32)
    out = _build_gather(s, b, vocab)(table, xt)
    return jnp.swapaxes(out[:, :embed].reshape(s, b, embed), 0, 1)
